# baseline (speedup 1.0000x reference)
import functools
import math

import jax
import jax.numpy as jnp
from jax import lax
from jax.experimental import pallas as pl
from jax.experimental.pallas import tpu as pltpu
from jax.experimental.pallas import tpu_sc as plsc

EMB = 64
SCALE = math.sqrt(EMB)
NC = 2
NS = 16
NW = NC * NS
CHUNK = 128
LANES = 16


def _sc_embed(tokens_w, table):
    n_chunks = tokens_w.shape[1]
    b_per_w = n_chunks * CHUNK
    B = NW * b_per_w
    mesh = plsc.VectorSubcoreMesh(core_axis_name="c", subcore_axis_name="s")

    @functools.partial(
        pl.kernel,
        out_type=jax.ShapeDtypeStruct((B, EMB), jnp.float32),
        mesh=mesh,
        scratch_types=[
            pltpu.VMEM((n_chunks, CHUNK), jnp.int32),
            pltpu.VMEM((CHUNK, EMB), jnp.float32),
            pltpu.SemaphoreType.DMA,
        ],
        compiler_params=pltpu.CompilerParams(use_tc_tiling_on_sc=False),
    )
    def body(tokens_hbm, table_hbm, out_hbm, idx_v, rows_v, sem):
        wid = lax.axis_index("s") * NC + lax.axis_index("c")
        base = wid * b_per_w
        pltpu.sync_copy(tokens_hbm.at[wid], idx_v)

        def chunk_body(j, carry):
            pltpu.async_copy(table_hbm.at[idx_v.at[j]], rows_v, sem).wait()

            def scale_row(r, carry2):
                for c in range(EMB // LANES):
                    sl = pl.ds(c * LANES, LANES)
                    rows_v[r, sl] = rows_v[r, sl] * SCALE
                return carry2

            lax.fori_loop(0, CHUNK, scale_row, 0, unroll=2)
            pltpu.sync_copy(rows_v, out_hbm.at[pl.ds(base + j * CHUNK, CHUNK)])
            return carry

        lax.fori_loop(0, n_chunks, chunk_body, 0)

    return body(tokens_w, table)


def kernel(tokens, table):
    R, C = tokens.shape
    B = R * C
    tokens_w = tokens.reshape(NW, B // (NW * CHUNK), CHUNK)
    out = _sc_embed(tokens_w, table)
    return out.reshape(R, C, EMB)


# speedup vs baseline: 1.0529x; 1.0529x over previous
import functools
import math

import jax
import jax.numpy as jnp
from jax import lax
from jax.experimental import pallas as pl
from jax.experimental.pallas import tpu as pltpu
from jax.experimental.pallas import tpu_sc as plsc

EMB = 64
SCALE = math.sqrt(EMB)
NC = 2
NS = 16
NW = NC * NS
CHUNK = 128
LANES = 16
NBUF = 4


def _sc_embed(tokens_w, table):
    n_chunks = tokens_w.shape[1]
    b_per_w = n_chunks * CHUNK
    B = NW * b_per_w
    mesh = plsc.VectorSubcoreMesh(core_axis_name="c", subcore_axis_name="s")

    @functools.partial(
        pl.kernel,
        out_type=jax.ShapeDtypeStruct((B, EMB), jnp.float32),
        mesh=mesh,
        scratch_types=[
            pltpu.VMEM((n_chunks, CHUNK), jnp.int32),
            pltpu.VMEM((NBUF, CHUNK, EMB), jnp.float32),
            pltpu.VMEM((NBUF, CHUNK, EMB), jnp.float32),
            pltpu.SemaphoreType.DMA((NBUF,)),
            pltpu.SemaphoreType.DMA((NBUF,)),
        ],
        compiler_params=pltpu.CompilerParams(use_tc_tiling_on_sc=False),
    )
    def body(tokens_hbm, table_hbm, out_hbm, idx_v, graw, sbuf, gsem, ssem):
        wid = lax.axis_index("s") * NC + lax.axis_index("c")
        base = wid * b_per_w
        pltpu.sync_copy(tokens_hbm.at[wid], idx_v)

        for b in range(NBUF):
            pltpu.async_copy(table_hbm.at[idx_v.at[b]], graw.at[b], gsem.at[b])

        def outer(g, carry):
            for b in range(NBUF):
                j = g * NBUF + b
                pltpu.make_async_copy(
                    table_hbm.at[idx_v.at[b]], graw.at[b], gsem.at[b]
                ).wait()
                @pl.when(j >= NBUF)
                def _():
                    pltpu.make_async_copy(
                        sbuf.at[b], out_hbm.at[pl.ds(base, CHUNK)], ssem.at[b]
                    ).wait()

                def scale_row(r, carry2):
                    for c in range(EMB // LANES):
                        sl = pl.ds(c * LANES, LANES)
                        sbuf[b, r, sl] = graw[b, r, sl] * SCALE
                    return carry2

                lax.fori_loop(0, CHUNK, scale_row, 0, unroll=4)

                pltpu.async_copy(
                    sbuf.at[b],
                    out_hbm.at[pl.ds(base + j * CHUNK, CHUNK)],
                    ssem.at[b],
                )

                @pl.when(j + NBUF < n_chunks)
                def _():
                    pltpu.async_copy(
                        table_hbm.at[idx_v.at[j + NBUF]], graw.at[b], gsem.at[b]
                    )
            return carry

        lax.fori_loop(0, n_chunks // NBUF, outer, 0)

        for b in range(NBUF):
            pltpu.make_async_copy(
                sbuf.at[b], out_hbm.at[pl.ds(base, CHUNK)], ssem.at[b]
            ).wait()

    return body(tokens_w, table)


def kernel(tokens, table):
    R, C = tokens.shape
    B = R * C
    tokens_w = tokens.reshape(NW, B // (NW * CHUNK), CHUNK)
    out = _sc_embed(tokens_w, table)
    return out.reshape(R, C, EMB)
